# baseline (speedup 1.0000x reference)
import jax
import jax.numpy as jnp
from jax.experimental import pallas as pl

_B, _V, _D, _P = 64, 321, 128, 12
_DP = _D * _P
_TW = 96
_M = _B * _V

_BM = 1024


def _head_kernel(xt_ref, xf_ref, w1_ref, w2_ref, b_ref, o_ref):
    acc = jnp.dot(xt_ref[...], w1_ref[...], preferred_element_type=jnp.float32)
    acc = acc + jnp.dot(xf_ref[...], w2_ref[...], preferred_element_type=jnp.float32)
    o_ref[...] = acc + b_ref[...]


def kernel(x_time, x_frequency, W, b):
    xt = x_time.reshape(_M, _DP)
    xf = x_frequency.reshape(_M, _DP)
    Wt = W.T
    w1 = Wt[:_DP]
    w2 = Wt[_DP:]
    b2 = b.reshape(1, _TW)
    out = pl.pallas_call(
        _head_kernel,
        grid=(pl.cdiv(_M, _BM),),
        in_specs=[
            pl.BlockSpec((_BM, _DP), lambda i: (i, 0)),
            pl.BlockSpec((_BM, _DP), lambda i: (i, 0)),
            pl.BlockSpec((_DP, _TW), lambda i: (0, 0)),
            pl.BlockSpec((_DP, _TW), lambda i: (0, 0)),
            pl.BlockSpec((1, _TW), lambda i: (0, 0)),
        ],
        out_specs=pl.BlockSpec((_BM, _TW), lambda i: (i, 0)),
        out_shape=jax.ShapeDtypeStruct((_M, _TW), jnp.float32),
    )(xt, xf, w1, w2, b2)
    return out.reshape(_B, _V, _TW)


# speedup vs baseline: 2.2928x; 2.2928x over previous
import jax
import jax.numpy as jnp
from jax.experimental import pallas as pl

_B, _V, _D, _P = 64, 321, 128, 12
_DP = _D * _P
_TW = 96
_M = _B * _V

_BM = 1024


def _head_kernel(xt_ref, xf_ref, w1_ref, w2_ref, b_ref, o_ref):
    bm = xt_ref.shape[0]
    xt = xt_ref[...].reshape(bm, _DP)
    xf = xf_ref[...].reshape(bm, _DP)
    acc = jnp.dot(xt, w1_ref[...], preferred_element_type=jnp.float32)
    acc = acc + jnp.dot(xf, w2_ref[...], preferred_element_type=jnp.float32)
    o_ref[...] = acc + b_ref[...]


def kernel(x_time, x_frequency, W, b):
    xt = x_time.transpose(0, 1, 3, 2).reshape(_M, _P, _D)
    xf = x_frequency.transpose(0, 1, 3, 2).reshape(_M, _P, _D)
    w1 = W[:, :_DP].reshape(_TW, _D, _P).transpose(0, 2, 1).reshape(_TW, _DP).T
    w2 = W[:, _DP:].reshape(_TW, _D, _P).transpose(0, 2, 1).reshape(_TW, _DP).T
    b2 = b.reshape(1, _TW)
    out = pl.pallas_call(
        _head_kernel,
        grid=(pl.cdiv(_M, _BM),),
        in_specs=[
            pl.BlockSpec((_BM, _P, _D), lambda i: (i, 0, 0)),
            pl.BlockSpec((_BM, _P, _D), lambda i: (i, 0, 0)),
            pl.BlockSpec((_DP, _TW), lambda i: (0, 0)),
            pl.BlockSpec((_DP, _TW), lambda i: (0, 0)),
            pl.BlockSpec((1, _TW), lambda i: (0, 0)),
        ],
        out_specs=pl.BlockSpec((_BM, _TW), lambda i: (i, 0)),
        out_shape=jax.ShapeDtypeStruct((_M, _TW), jnp.float32),
    )(xt, xf, w1, w2, b2)
    return out.reshape(_B, _V, _TW)


# speedup vs baseline: 5.7357x; 2.5017x over previous
import jax
import jax.numpy as jnp
from jax.experimental import pallas as pl

_B, _V, _D, _P = 64, 321, 128, 12
_DP = _D * _P
_TW = 96

_VB = 8


def _head_kernel(xt_ref, xf_ref, w1_ref, w2_ref, b_ref, o_ref):
    for vi in range(_VB):
        acc = jnp.broadcast_to(b_ref[...], (_B, _TW)).astype(jnp.float32)
        for p in range(_P):
            acc = acc + jnp.dot(xt_ref[vi, p], w1_ref[p],
                                preferred_element_type=jnp.float32)
            acc = acc + jnp.dot(xf_ref[vi, p], w2_ref[p],
                                preferred_element_type=jnp.float32)
        o_ref[vi] = acc


def kernel(x_time, x_frequency, W, b):
    xt = x_time.transpose(1, 3, 0, 2)
    xf = x_frequency.transpose(1, 3, 0, 2)
    w1 = W[:, :_DP].reshape(_TW, _D, _P).transpose(2, 1, 0)
    w2 = W[:, _DP:].reshape(_TW, _D, _P).transpose(2, 1, 0)
    b2 = b.reshape(1, _TW)
    out = pl.pallas_call(
        _head_kernel,
        grid=(pl.cdiv(_V, _VB),),
        in_specs=[
            pl.BlockSpec((_VB, _P, _B, _D), lambda i: (i, 0, 0, 0)),
            pl.BlockSpec((_VB, _P, _B, _D), lambda i: (i, 0, 0, 0)),
            pl.BlockSpec((_P, _D, _TW), lambda i: (0, 0, 0)),
            pl.BlockSpec((_P, _D, _TW), lambda i: (0, 0, 0)),
            pl.BlockSpec((1, _TW), lambda i: (0, 0)),
        ],
        out_specs=pl.BlockSpec((_VB, _B, _TW), lambda i: (i, 0, 0)),
        out_shape=jax.ShapeDtypeStruct((_V, _B, _TW), jnp.float32),
    )(xt, xf, w1, w2, b2)
    return out.transpose(1, 0, 2)


# speedup vs baseline: 6.6372x; 1.1572x over previous
import jax
import jax.numpy as jnp
from jax.experimental import pallas as pl

_B, _V, _D, _P = 64, 321, 128, 12
_DP = _D * _P
_TW = 96

_VB = 8


def _head_kernel(xt_ref, xf_ref, w1_ref, w2_ref, b_ref, o_ref):
    accs = []
    for vi in range(_VB):
        acc = jnp.broadcast_to(b_ref[...], (_B, _TW)).astype(jnp.float32)
        for p in range(_P):
            acc = acc + jnp.dot(xt_ref[vi, p], w1_ref[p],
                                preferred_element_type=jnp.float32)
            acc = acc + jnp.dot(xf_ref[vi, p], w2_ref[p],
                                preferred_element_type=jnp.float32)
        accs.append(acc)
    o_ref[...] = jnp.stack(accs, axis=0).transpose(1, 0, 2)


def kernel(x_time, x_frequency, W, b):
    xt = x_time.transpose(1, 3, 0, 2)
    xf = x_frequency.transpose(1, 3, 0, 2)
    w1 = W[:, :_DP].reshape(_TW, _D, _P).transpose(2, 1, 0)
    w2 = W[:, _DP:].reshape(_TW, _D, _P).transpose(2, 1, 0)
    b2 = b.reshape(1, _TW)
    out = pl.pallas_call(
        _head_kernel,
        grid=(pl.cdiv(_V, _VB),),
        in_specs=[
            pl.BlockSpec((_VB, _P, _B, _D), lambda i: (i, 0, 0, 0)),
            pl.BlockSpec((_VB, _P, _B, _D), lambda i: (i, 0, 0, 0)),
            pl.BlockSpec((_P, _D, _TW), lambda i: (0, 0, 0)),
            pl.BlockSpec((_P, _D, _TW), lambda i: (0, 0, 0)),
            pl.BlockSpec((1, _TW), lambda i: (0, 0)),
        ],
        out_specs=pl.BlockSpec((_B, _VB, _TW), lambda i: (0, i, 0)),
        out_shape=jax.ShapeDtypeStruct((_B, _V, _TW), jnp.float32),
    )(xt, xf, w1, w2, b2)
    return out


# speedup vs baseline: 7.2313x; 1.0895x over previous
import jax
import jax.numpy as jnp
from jax.experimental import pallas as pl

_B, _V, _D, _P = 64, 321, 128, 12
_DP = _D * _P
_TW = 96

_VB = 16


def _head_kernel(xt_ref, xf_ref, w1_ref, w2_ref, b_ref, o_ref):
    accs = []
    for vi in range(_VB):
        acc = jnp.broadcast_to(b_ref[...], (_B, _TW)).astype(jnp.float32)
        for p in range(_P):
            acc = acc + jnp.dot(xt_ref[vi, p], w1_ref[p],
                                preferred_element_type=jnp.float32)
            acc = acc + jnp.dot(xf_ref[vi, p], w2_ref[p],
                                preferred_element_type=jnp.float32)
        accs.append(acc)
    o_ref[...] = jnp.stack(accs, axis=0).transpose(1, 0, 2)


def kernel(x_time, x_frequency, W, b):
    xt = x_time.transpose(1, 3, 0, 2)
    xf = x_frequency.transpose(1, 3, 0, 2)
    w1 = W[:, :_DP].reshape(_TW, _D, _P).transpose(2, 1, 0)
    w2 = W[:, _DP:].reshape(_TW, _D, _P).transpose(2, 1, 0)
    b2 = b.reshape(1, _TW)
    out = pl.pallas_call(
        _head_kernel,
        grid=(pl.cdiv(_V, _VB),),
        in_specs=[
            pl.BlockSpec((_VB, _P, _B, _D), lambda i: (i, 0, 0, 0)),
            pl.BlockSpec((_VB, _P, _B, _D), lambda i: (i, 0, 0, 0)),
            pl.BlockSpec((_P, _D, _TW), lambda i: (0, 0, 0)),
            pl.BlockSpec((_P, _D, _TW), lambda i: (0, 0, 0)),
            pl.BlockSpec((1, _TW), lambda i: (0, 0)),
        ],
        out_specs=pl.BlockSpec((_B, _VB, _TW), lambda i: (0, i, 0)),
        out_shape=jax.ShapeDtypeStruct((_B, _V, _TW), jnp.float32),
    )(xt, xf, w1, w2, b2)
    return out


# speedup vs baseline: 7.2343x; 1.0004x over previous
import jax
import jax.numpy as jnp
from jax.experimental import pallas as pl
from jax.experimental.pallas import tpu as pltpu

_B, _V, _D, _P = 64, 321, 128, 12
_DP = _D * _P
_TW = 96

_VB = 16


def _head_kernel(xt_ref, xf_ref, w1_ref, w2_ref, b_ref, o_ref):
    accs = []
    for vi in range(_VB):
        acc = jnp.broadcast_to(b_ref[...], (_B, _TW)).astype(jnp.float32)
        for p in range(_P):
            acc = acc + jnp.dot(xt_ref[vi, p], w1_ref[p],
                                preferred_element_type=jnp.float32)
            acc = acc + jnp.dot(xf_ref[vi, p], w2_ref[p],
                                preferred_element_type=jnp.float32)
        accs.append(acc)
    o_ref[...] = jnp.stack(accs, axis=0).transpose(1, 0, 2)


def kernel(x_time, x_frequency, W, b):
    xt = x_time.transpose(1, 3, 0, 2)
    xf = x_frequency.transpose(1, 3, 0, 2)
    w1 = W[:, :_DP].reshape(_TW, _D, _P).transpose(2, 1, 0)
    w2 = W[:, _DP:].reshape(_TW, _D, _P).transpose(2, 1, 0)
    b2 = b.reshape(1, _TW)
    out = pl.pallas_call(
        _head_kernel,
        grid=(pl.cdiv(_V, _VB),),
        in_specs=[
            pl.BlockSpec((_VB, _P, _B, _D), lambda i: (i, 0, 0, 0)),
            pl.BlockSpec((_VB, _P, _B, _D), lambda i: (i, 0, 0, 0)),
            pl.BlockSpec((_P, _D, _TW), lambda i: (0, 0, 0)),
            pl.BlockSpec((_P, _D, _TW), lambda i: (0, 0, 0)),
            pl.BlockSpec((1, _TW), lambda i: (0, 0)),
        ],
        out_specs=pl.BlockSpec((_B, _VB, _TW), lambda i: (0, i, 0)),
        out_shape=jax.ShapeDtypeStruct((_B, _V, _TW), jnp.float32),
        compiler_params=pltpu.CompilerParams(
            dimension_semantics=("parallel",),
        ),
    )(xt, xf, w1, w2, b2)
    return out
